# baseline (speedup 1.0000x reference)
import functools

import jax
import jax.numpy as jnp
from jax import lax
from jax.experimental import pallas as pl
from jax.experimental.pallas import tpu as pltpu
from jax.experimental.pallas import tpu_sc as plsc

_EMBED = 64
_NW = 32
_CHUNK = 128
_NBUF = 4
_LANES = 16
_SCALE = 8.0


def _body(x3, table, out, idx_v, gbufs, sbufs, *sems):
    gsem = sems[:_NBUF]
    ssem = sems[_NBUF:]
    n_chunks = idx_v.shape[0]

    cid = lax.axis_index("c")
    sid = lax.axis_index("s")
    wid = cid * 16 + sid
    base = wid * n_chunks

    pltpu.sync_copy(x3.at[pl.ds(base, n_chunks)], idx_v)

    def gather(c, b):
        return pltpu.make_async_copy(table.at[idx_v.at[c]], gbufs.at[b], gsem[b])

    def scatter(c, b):
        return pltpu.make_async_copy(sbufs.at[b], out.at[base + c], ssem[b])

    for b in range(_NBUF):
        gather(b, b).start()

    def round_body(g, _):
        for b in range(_NBUF):
            c = g * _NBUF + b
            gather(c, b).wait()

            @pl.when(g > 0)
            def _():
                scatter(c - _NBUF, b).wait()

            def scale_rows(r, _):
                for rr in range(4):
                    row = r * 4 + rr
                    for j in range(_EMBED // _LANES):
                        sl = pl.ds(j * _LANES, _LANES)
                        sbufs[b, row, sl] = gbufs[b, row, sl] * _SCALE
                return 0

            lax.fori_loop(0, _CHUNK // 4, scale_rows, 0)
            scatter(c, b).start()

            @pl.when(c + _NBUF < n_chunks)
            def _():
                gather(c + _NBUF, b).start()
        return 0

    lax.fori_loop(0, n_chunks // _NBUF, round_body, 0)

    for b in range(_NBUF):
        scatter(n_chunks - _NBUF + b, b).wait()


def kernel(x, embedding):
    batch, hist = x.shape
    n = batch * hist
    assert n % (_NW * _CHUNK) == 0
    n_blocks = n // _CHUNK
    n_chunks = n_blocks // _NW

    x3 = x.reshape(n_blocks, _CHUNK)

    mesh = plsc.VectorSubcoreMesh(core_axis_name="c", subcore_axis_name="s")
    run = pl.kernel(
        _body,
        out_type=jax.ShapeDtypeStruct((n_blocks, _CHUNK, _EMBED), jnp.float32),
        mesh=mesh,
        scratch_types=[
            pltpu.VMEM((n_chunks, _CHUNK), jnp.int32),
            pltpu.VMEM((_NBUF, _CHUNK, _EMBED), jnp.float32),
            pltpu.VMEM((_NBUF, _CHUNK, _EMBED), jnp.float32),
        ] + [pltpu.SemaphoreType.DMA] * (2 * _NBUF),
        compiler_params=pltpu.CompilerParams(use_tc_tiling_on_sc=False),
    )
    out = run(x3, embedding)
    return out.reshape(batch, hist, _EMBED)
